# baseline (speedup 1.0000x reference)
import functools

import jax
import jax.numpy as jnp
from jax import lax
from jax.experimental import pallas as pl
from jax.experimental.pallas import tpu as pltpu
from jax.experimental.pallas import tpu_sc as plsc

_COVALENT_RADII = [
    0.2, 0.31, 0.28, 1.28, 0.96, 0.84, 0.76, 0.71, 0.66, 0.57, 0.58, 1.66,
    1.41, 1.21, 1.11, 1.07, 1.05, 1.02, 1.06, 2.03, 1.76, 1.70, 1.60, 1.53,
    1.39, 1.39, 1.32, 1.26, 1.24, 1.32, 1.22, 1.22, 1.20, 1.19, 1.20, 1.20,
    1.16, 2.20, 1.95, 1.90, 1.75, 1.64, 1.54, 1.47, 1.46, 1.42, 1.39, 1.45,
    1.44, 1.42, 1.39, 1.39, 1.38, 1.39, 1.40, 2.44, 2.15, 2.07, 2.04, 2.03,
    2.01, 1.99, 1.98, 1.98, 1.96, 1.94, 1.92, 1.92, 1.89, 1.90, 1.87, 1.87,
    1.75, 1.70, 1.62, 1.51, 1.44, 1.41, 1.36, 1.36, 1.32, 1.45, 1.46, 1.48,
    1.40, 1.50, 1.50, 2.60, 2.21, 2.15, 2.06, 2.00, 1.96,
]
_C0, _C1, _C2, _C3 = 0.1818, 0.5099, 0.2802, 0.02817

_NC, _NS = 2, 16
_NW = _NC * _NS
_PACKG = 128
_BR = 800


def _round_up(v, m):
    return (v + m - 1) // m * m


@functools.partial(jax.jit, static_argnames=("n_real", "np_pad"))
def _zbl_sc(x, edge_index, node_attrs, ainv_tab, zfac_tab, rminv_tab,
            *, n_real, np_pad):
    E = x.shape[0]
    NE = node_attrs.shape[1]
    NP = np_pad
    SLICE = NP // _NW
    PSL = SLICE // 8
    CH = 2048
    NCHT = E // CH
    NCHQ, REM = divmod(NCHT, _NW)
    MAXCH = NCHQ + (1 if REM else 0)
    MAXCH += MAXCH % 2
    RSL = NP // _NS
    RSEG = 1600 if RSL % 1600 == 0 else RSL
    NSEG = RSL // RSEG

    NBLK = NP // _BR
    NIN = node_attrs.shape[0] // _BR

    def am_body(a_ref, o_ref):
        a = a_ref[...]
        best = a[:, 0:1]
        idx = jnp.zeros((_BR, 1), jnp.int32)
        for j in range(1, NE):
            vj = a[:, j:j + 1]
            m = vj > best
            idx = jnp.where(m, j, idx)
            best = jnp.where(m, vj, best)
        o_ref[...] = jnp.bitwise_or(idx, 0x40000000)

    elems2d = pl.pallas_call(
        am_body,
        grid=(NBLK,),
        in_specs=[pl.BlockSpec((_BR, NE),
                               lambda i: (jnp.minimum(i, NIN - 1), 0))],
        out_specs=pl.BlockSpec((_BR, 1), lambda i: (i, 0)),
        out_shape=jax.ShapeDtypeStruct((NP, 1), jnp.int32),
    )(node_attrs)
    elems_f = lax.bitcast_convert_type(elems2d.reshape(NP), jnp.float32)
    x_flat = x.reshape(E)

    mesh = plsc.VectorSubcoreMesh(core_axis_name="c", subcore_axis_name="s",
                                  num_cores=_NC, num_subcores=_NS)

    def body(x_hbm, ei_hbm, elems_hbm, ainv_hbm, zfac_hbm, rminv_hbm,
             out_hbm, acc, packed, sr_buf, e_buf, x_buf,
             ainv_t, zfac_t, rminv_t, hbm_packed, hbm_parts, sem0, sem1):
        cid = lax.axis_index("c")
        sid = lax.axis_index("s")
        wid = sid * _NC + cid

        iota = lax.iota(jnp.int32, 16)
        iota8 = iota * 8

        pltpu.sync_copy(ainv_hbm, ainv_t)
        pltpu.sync_copy(zfac_hbm, zfac_t)
        pltpu.sync_copy(rminv_hbm, rminv_t)

        zf16 = jnp.zeros((16,), jnp.float32)

        @pl.loop(0, NP // 16)
        def _(i):
            acc[pl.ds(i * 16, 16)] = zf16

        for soff in (0, _NS):
            S = sid + soff

            @pl.loop(0, SLICE // 640)
            def _(pr, S=S):
                base = S * SLICE + pr * 640
                pltpu.sync_copy(elems_hbm.at[pl.ds(base, 640)], e_buf)

                @pl.loop(0, 640 // _PACKG)
                def _(it, S=S, pr=pr):
                    nbase = it * _PACKG
                    w = jnp.zeros((16,), jnp.int32)
                    for k in range(8):
                        ef = plsc.load_gather(e_buf, [iota8 + (nbase + k)])
                        ek = jnp.bitwise_and(plsc.bitcast(ef, jnp.int32), 15)
                        w = jnp.bitwise_or(w, jnp.left_shift(ek, 4 * k))
                    packed[pl.ds(S * PSL + pr * 80 + it * 16, 16)] = w

        NPACK = NP // 8
        pbase = cid * NPACK
        for soff in (0, _NS):
            S = sid + soff
            pltpu.sync_copy(packed.at[pl.ds(S * PSL, PSL)],
                            hbm_packed.at[pl.ds(pbase + S * PSL, PSL)])
        plsc.subcore_barrier()
        pltpu.sync_copy(hbm_packed.at[pl.ds(pbase, NPACK)], packed)

        nch = jnp.where(wid < REM, NCHQ + 1, NCHQ)
        c0 = NCHQ * wid + jnp.minimum(wid, REM)

        def start(ci, slot, sem):
            b = (c0 + ci) * CH
            pltpu.async_copy(ei_hbm.at[:, pl.ds(b, CH)],
                             sr_buf.at[pl.ds(2 * slot, 2), :], sem)
            pltpu.async_copy(x_hbm.at[pl.ds(b, CH)],
                             x_buf.at[pl.ds(slot * CH, CH)], sem)

        def wait(ci, slot, sem):
            b = (c0 + ci) * CH
            pltpu.make_async_copy(ei_hbm.at[:, pl.ds(b, CH)],
                                  sr_buf.at[pl.ds(2 * slot, 2), :], sem).wait()
            pltpu.make_async_copy(x_hbm.at[pl.ds(b, CH)],
                                  x_buf.at[pl.ds(slot * CH, CH)], sem).wait()

        start(0, 0, sem0)

        @pl.loop(0, MAXCH, step=2)
        def _(ci):
            for bslot in (0, 1):
                sem = sem0 if bslot == 0 else sem1
                osem = sem1 if bslot == 0 else sem0
                cur = ci + bslot

                @pl.when(cur < nch)
                def _(cur=cur, bslot=bslot, sem=sem, osem=osem):
                    @pl.when(cur + 1 < nch)
                    def _():
                        start(cur + 1, 1 - bslot, osem)

                    wait(cur, bslot, sem)

                @pl.when(cur < nch)
                def _(cur=cur, bslot=bslot):
                    @plsc.parallel_loop(0, CH // 16, unroll=4)
                    def _(i, bslot=bslot):
                        sl = pl.ds(i * 16, 16)
                        s = sr_buf[2 * bslot, sl]
                        r = sr_buf[2 * bslot + 1, sl]
                        xv = x_buf[pl.ds(bslot * CH + i * 16, 16)]
                        ws = plsc.load_gather(packed, [lax.shift_right_logical(s, 3)])
                        wr = plsc.load_gather(packed, [lax.shift_right_logical(r, 3)])
                        shs = jnp.left_shift(jnp.bitwise_and(s, 7), 2)
                        shr = jnp.left_shift(jnp.bitwise_and(r, 7), 2)
                        eu = jnp.bitwise_and(lax.shift_right_logical(ws, shs), 15)
                        ev = jnp.bitwise_and(lax.shift_right_logical(wr, shr), 15)
                        p = jnp.bitwise_or(jnp.left_shift(eu, 4), ev)
                        ainv = plsc.load_gather(ainv_t, [p])
                        zfac = plsc.load_gather(zfac_t, [p])
                        rminv = plsc.load_gather(rminv_t, [p])
                        roa = xv * ainv
                        phi = (_C0 * jnp.exp(roa * -3.2)
                               + _C1 * jnp.exp(roa * -0.9423)
                               + _C2 * jnp.exp(roa * -0.4028)
                               + _C3 * jnp.exp(roa * -0.2016))
                        t = xv * rminv
                        t2 = t * t
                        t6 = t2 * t2 * t2
                        env = 1.0 + t6 * (-28.0 + t * (48.0 - 21.0 * t))
                        env = jnp.maximum(env, 0.0)
                        v = zfac * phi / xv * env
                        plsc.addupdate_scatter(acc, [r], v)

        rbase = sid * RSL
        pltpu.sync_copy(acc,
                        hbm_parts.at[pl.ds((cid * _NS + sid) * NP, NP)])
        plsc.subcore_barrier()

        @pl.loop(0, RSL // 16)
        def _(i):
            acc[pl.ds(rbase + i * 16, 16)] = zf16

        NRR = _NS * NSEG

        def rsrc(j):
            k = j // NSEG
            sg = j % NSEG
            return (cid * _NS + k) * NP + rbase + sg * RSEG, sg

        def rstart(j, slot, sem):
            off, _ = rsrc(j)
            pltpu.async_copy(hbm_parts.at[pl.ds(off, RSEG)],
                             x_buf.at[pl.ds(slot * RSEG, RSEG)], sem)

        def rwait(j, slot, sem):
            off, _ = rsrc(j)
            pltpu.make_async_copy(hbm_parts.at[pl.ds(off, RSEG)],
                                  x_buf.at[pl.ds(slot * RSEG, RSEG)],
                                  sem).wait()

        rstart(0, 0, sem0)

        @pl.loop(0, NRR, step=2)
        def _(j):
            for bslot in (0, 1):
                sem = sem0 if bslot == 0 else sem1
                osem = sem1 if bslot == 0 else sem0
                cur = j + bslot

                @pl.when(cur + 1 < NRR)
                def _(cur=cur, bslot=bslot, osem=osem):
                    rstart(cur + 1, 1 - bslot, osem)

                rwait(cur, bslot, sem)
                _, sg = rsrc(cur)
                segbase = rbase + sg * RSEG

                @pl.loop(0, RSEG // 16, unroll=4)
                def _(i, segbase=segbase, bslot=bslot):
                    d = pl.ds(segbase + i * 16, 16)
                    acc[d] = acc[d] + x_buf[pl.ds(bslot * RSEG + i * 16, 16)]

        pltpu.sync_copy(acc.at[pl.ds(rbase, RSL)],
                        out_hbm.at[pl.ds(cid * NP + rbase, RSL)])

    partials = pl.kernel(
        body,
        out_type=jax.ShapeDtypeStruct((2 * NP,), jnp.float32),
        mesh=mesh,
        compiler_params=pltpu.CompilerParams(needs_layout_passes=False),
        scratch_types=[
            pltpu.VMEM((NP,), jnp.float32),
            pltpu.VMEM((NP // 8,), jnp.int32),
            pltpu.VMEM((4, CH), jnp.int32),
            pltpu.VMEM((640,), jnp.float32),
            pltpu.VMEM((2 * CH,), jnp.float32),
            pltpu.VMEM((160,), jnp.float32),
            pltpu.VMEM((160,), jnp.float32),
            pltpu.VMEM((160,), jnp.float32),
            pltpu.HBM((2 * (NP // 8),), jnp.int32),
            pltpu.HBM((2 * _NS * NP,), jnp.float32),
            pltpu.SemaphoreType.DMA,
            pltpu.SemaphoreType.DMA,
        ],
    )(x_flat, edge_index, elems_f, ainv_tab, zfac_tab, rminv_tab)

    pr = partials.reshape(2, NP // 128, 128)
    rows = NP // 128

    def add_body(a_ref, o_ref):
        o_ref[...] = a_ref[0] + a_ref[1]

    blk = 80 if rows % 80 == 0 else 8
    out = pl.pallas_call(
        add_body,
        grid=(rows // blk,),
        in_specs=[pl.BlockSpec((2, blk, 128), lambda i: (0, i, 0))],
        out_specs=pl.BlockSpec((blk, 128), lambda i: (i, 0)),
        out_shape=jax.ShapeDtypeStruct((rows, 128), jnp.float32),
    )(pr)
    return out.reshape(NP)[:n_real]


def kernel(x, node_attrs, edge_index, atomic_numbers):
    N, NE = node_attrs.shape
    E = x.shape[0]

    z = atomic_numbers.astype(jnp.float32)
    pw = jnp.power(z, jnp.float32(0.3))
    radii = jnp.asarray(_COVALENT_RADII, jnp.float32)
    rad = radii[atomic_numbers]
    ii = jnp.minimum(jnp.arange(16), NE - 1)
    pwp = pw[ii]
    zp = z[ii]
    radp = rad[ii]
    ainv_tab = ((pwp[:, None] + pwp[None, :]) / (0.4543 * 0.529)).astype(jnp.float32).reshape(256)[:160]
    zfac_tab = (0.5 * 14.3996 * zp[:, None] * zp[None, :]).astype(jnp.float32).reshape(256)[:160]
    rminv_tab = (1.0 / (radp[:, None] + radp[None, :])).astype(jnp.float32).reshape(256)[:160]

    attrs_in = node_attrs
    if N % _BR:
        attrs_in = jnp.pad(node_attrs, ((0, _BR - N % _BR), (0, 0)))
    NP = _round_up(_round_up(N, _BR), _NW * 640)

    return _zbl_sc(x, edge_index, attrs_in, ainv_tab,
                   zfac_tab, rminv_tab, n_real=N, np_pad=NP)


# speedup vs baseline: 1.5515x; 1.5515x over previous
import functools

import jax
import jax.numpy as jnp
from jax import lax
from jax.experimental import pallas as pl
from jax.experimental.pallas import tpu as pltpu
from jax.experimental.pallas import tpu_sc as plsc

_COVALENT_RADII = [
    0.2, 0.31, 0.28, 1.28, 0.96, 0.84, 0.76, 0.71, 0.66, 0.57, 0.58, 1.66,
    1.41, 1.21, 1.11, 1.07, 1.05, 1.02, 1.06, 2.03, 1.76, 1.70, 1.60, 1.53,
    1.39, 1.39, 1.32, 1.26, 1.24, 1.32, 1.22, 1.22, 1.20, 1.19, 1.20, 1.20,
    1.16, 2.20, 1.95, 1.90, 1.75, 1.64, 1.54, 1.47, 1.46, 1.42, 1.39, 1.45,
    1.44, 1.42, 1.39, 1.39, 1.38, 1.39, 1.40, 2.44, 2.15, 2.07, 2.04, 2.03,
    2.01, 1.99, 1.98, 1.98, 1.96, 1.94, 1.92, 1.92, 1.89, 1.90, 1.87, 1.87,
    1.75, 1.70, 1.62, 1.51, 1.44, 1.41, 1.36, 1.36, 1.32, 1.45, 1.46, 1.48,
    1.40, 1.50, 1.50, 2.60, 2.21, 2.15, 2.06, 2.00, 1.96,
]
_C0, _C1, _C2, _C3 = 0.1818, 0.5099, 0.2802, 0.02817

_NC, _NS = 2, 16
_NW = _NC * _NS
_PACKG = 128
_BR = 800


def _round_up(v, m):
    return (v + m - 1) // m * m


@functools.partial(jax.jit, static_argnames=("n_real", "np_pad"))
def _zbl_sc(x, edge_index, node_attrs, ainv_tab, zfac_tab, rminv_tab,
            *, n_real, np_pad):
    E = x.shape[0]
    NE = node_attrs.shape[1]
    NP = np_pad
    SLICE = NP // _NW
    PSL = SLICE // 8
    CH = 2048
    NCHT = E // CH
    NCHQ, REM = divmod(NCHT, _NW)
    MAXCH = NCHQ + (1 if REM else 0)
    MAXCH += MAXCH % 2
    RSL = NP // _NS
    RSEG = 1600 if RSL % 1600 == 0 else RSL
    NSEG = RSL // RSEG

    N = node_attrs.shape[0]
    attrs_flat = node_attrs.reshape(N * NE)
    x_flat = x.reshape(E)

    mesh = plsc.VectorSubcoreMesh(core_axis_name="c", subcore_axis_name="s",
                                  num_cores=_NC, num_subcores=_NS)

    def body(x_hbm, ei_hbm, attrs_hbm, ainv_hbm, zfac_hbm, rminv_hbm,
             out_hbm, acc, packed, sr_buf, e_buf, x_buf,
             ainv_t, zfac_t, rminv_t, hbm_packed, hbm_parts, sem0, sem1):
        cid = lax.axis_index("c")
        sid = lax.axis_index("s")
        wid = sid * _NC + cid

        iota = lax.iota(jnp.int32, 16)
        iota8 = iota * 8
        iota10 = iota * 10

        pltpu.sync_copy(ainv_hbm, ainv_t)
        pltpu.sync_copy(zfac_hbm, zfac_t)
        pltpu.sync_copy(rminv_hbm, rminv_t)

        zf16 = jnp.zeros((16,), jnp.float32)

        @pl.loop(0, NP // 16)
        def _(i):
            acc[pl.ds(i * 16, 16)] = zf16

        for soff in (0, _NS):
            S = sid + soff

            @pl.loop(0, SLICE // 640)
            def _(pr, S=S):
                nodebase = S * SLICE + pr * 640
                for q in (0, 1, 2, 3):
                    nb = nodebase + q * 160

                    @pl.when(nb + 160 <= N)
                    def _(nb=nb, q=q):
                        pltpu.sync_copy(attrs_hbm.at[pl.ds(nb * 10, 1600)],
                                        x_buf.at[pl.ds(0, 1600)])

                        @pl.loop(0, 10, unroll=2)
                        def _(g, q=q):
                            off = g * 160
                            best = plsc.load_gather(x_buf, [iota10 + off])
                            eidx = jnp.zeros((16,), jnp.int32)
                            for j in range(1, 10):
                                vj = plsc.load_gather(x_buf, [iota10 + (off + j)])
                                m = vj > best
                                eidx = jnp.where(m, j, eidx)
                                best = jnp.where(m, vj, best)
                            e_buf[pl.ds(q * 160 + g * 16, 16)] = eidx

                @pl.loop(0, 640 // _PACKG)
                def _(it, S=S, pr=pr):
                    nbase = it * _PACKG
                    w = jnp.zeros((16,), jnp.int32)
                    for k in range(8):
                        ek = plsc.load_gather(e_buf, [iota8 + (nbase + k)])
                        w = jnp.bitwise_or(w, jnp.left_shift(ek, 4 * k))
                    packed[pl.ds(S * PSL + pr * 80 + it * 16, 16)] = w

        NPACK = NP // 8
        pbase = cid * NPACK
        for soff in (0, _NS):
            S = sid + soff
            pltpu.sync_copy(packed.at[pl.ds(S * PSL, PSL)],
                            hbm_packed.at[pl.ds(pbase + S * PSL, PSL)])
        plsc.subcore_barrier()
        pltpu.sync_copy(hbm_packed.at[pl.ds(pbase, NPACK)], packed)

        nch = jnp.where(wid < REM, NCHQ + 1, NCHQ)
        c0 = NCHQ * wid + jnp.minimum(wid, REM)

        def start(ci, slot, sem):
            b = (c0 + ci) * CH
            pltpu.async_copy(ei_hbm.at[:, pl.ds(b, CH)],
                             sr_buf.at[pl.ds(2 * slot, 2), :], sem)
            pltpu.async_copy(x_hbm.at[pl.ds(b, CH)],
                             x_buf.at[pl.ds(slot * CH, CH)], sem)

        def wait(ci, slot, sem):
            b = (c0 + ci) * CH
            pltpu.make_async_copy(ei_hbm.at[:, pl.ds(b, CH)],
                                  sr_buf.at[pl.ds(2 * slot, 2), :], sem).wait()
            pltpu.make_async_copy(x_hbm.at[pl.ds(b, CH)],
                                  x_buf.at[pl.ds(slot * CH, CH)], sem).wait()

        start(0, 0, sem0)

        @pl.loop(0, MAXCH, step=2)
        def _(ci):
            for bslot in (0, 1):
                sem = sem0 if bslot == 0 else sem1
                osem = sem1 if bslot == 0 else sem0
                cur = ci + bslot

                @pl.when(cur < nch)
                def _(cur=cur, bslot=bslot, sem=sem, osem=osem):
                    @pl.when(cur + 1 < nch)
                    def _():
                        start(cur + 1, 1 - bslot, osem)

                    wait(cur, bslot, sem)

                @pl.when(cur < nch)
                def _(cur=cur, bslot=bslot):
                    @plsc.parallel_loop(0, CH // 16, unroll=4)
                    def _(i, bslot=bslot):
                        sl = pl.ds(i * 16, 16)
                        s = sr_buf[2 * bslot, sl]
                        r = sr_buf[2 * bslot + 1, sl]
                        xv = x_buf[pl.ds(bslot * CH + i * 16, 16)]
                        ws = plsc.load_gather(packed, [lax.shift_right_logical(s, 3)])
                        wr = plsc.load_gather(packed, [lax.shift_right_logical(r, 3)])
                        shs = jnp.left_shift(jnp.bitwise_and(s, 7), 2)
                        shr = jnp.left_shift(jnp.bitwise_and(r, 7), 2)
                        eu = jnp.bitwise_and(lax.shift_right_logical(ws, shs), 15)
                        ev = jnp.bitwise_and(lax.shift_right_logical(wr, shr), 15)
                        p = jnp.bitwise_or(jnp.left_shift(eu, 4), ev)
                        ainv = plsc.load_gather(ainv_t, [p])
                        zfac = plsc.load_gather(zfac_t, [p])
                        rminv = plsc.load_gather(rminv_t, [p])
                        roa = xv * ainv
                        phi = (_C0 * jnp.exp(roa * -3.2)
                               + _C1 * jnp.exp(roa * -0.9423)
                               + _C2 * jnp.exp(roa * -0.4028)
                               + _C3 * jnp.exp(roa * -0.2016))
                        t = xv * rminv
                        t2 = t * t
                        t6 = t2 * t2 * t2
                        env = 1.0 + t6 * (-28.0 + t * (48.0 - 21.0 * t))
                        env = jnp.maximum(env, 0.0)
                        v = zfac * phi / xv * env
                        plsc.addupdate_scatter(acc, [r], v)

        rbase = sid * RSL
        pltpu.sync_copy(acc,
                        hbm_parts.at[pl.ds((cid * _NS + sid) * NP, NP)])
        plsc.subcore_barrier()

        @pl.loop(0, RSL // 16)
        def _(i):
            acc[pl.ds(rbase + i * 16, 16)] = zf16

        NRR = _NS * NSEG

        def rsrc(j):
            k = j // NSEG
            sg = j % NSEG
            return (cid * _NS + k) * NP + rbase + sg * RSEG, sg

        def rstart(j, slot, sem):
            off, _ = rsrc(j)
            pltpu.async_copy(hbm_parts.at[pl.ds(off, RSEG)],
                             x_buf.at[pl.ds(slot * RSEG, RSEG)], sem)

        def rwait(j, slot, sem):
            off, _ = rsrc(j)
            pltpu.make_async_copy(hbm_parts.at[pl.ds(off, RSEG)],
                                  x_buf.at[pl.ds(slot * RSEG, RSEG)],
                                  sem).wait()

        rstart(0, 0, sem0)

        @pl.loop(0, NRR, step=2)
        def _(j):
            for bslot in (0, 1):
                sem = sem0 if bslot == 0 else sem1
                osem = sem1 if bslot == 0 else sem0
                cur = j + bslot

                @pl.when(cur + 1 < NRR)
                def _(cur=cur, bslot=bslot, osem=osem):
                    rstart(cur + 1, 1 - bslot, osem)

                rwait(cur, bslot, sem)
                _, sg = rsrc(cur)
                segbase = rbase + sg * RSEG

                @pl.loop(0, RSEG // 16, unroll=4)
                def _(i, segbase=segbase, bslot=bslot):
                    d = pl.ds(segbase + i * 16, 16)
                    acc[d] = acc[d] + x_buf[pl.ds(bslot * RSEG + i * 16, 16)]

        pltpu.sync_copy(acc.at[pl.ds(rbase, RSL)],
                        out_hbm.at[pl.ds(cid * NP + rbase, RSL)])

    partials = pl.kernel(
        body,
        out_type=jax.ShapeDtypeStruct((2 * NP,), jnp.float32),
        mesh=mesh,
        compiler_params=pltpu.CompilerParams(needs_layout_passes=False),
        scratch_types=[
            pltpu.VMEM((NP,), jnp.float32),
            pltpu.VMEM((NP // 8,), jnp.int32),
            pltpu.VMEM((4, CH), jnp.int32),
            pltpu.VMEM((640,), jnp.int32),
            pltpu.VMEM((2 * CH,), jnp.float32),
            pltpu.VMEM((160,), jnp.float32),
            pltpu.VMEM((160,), jnp.float32),
            pltpu.VMEM((160,), jnp.float32),
            pltpu.HBM((2 * (NP // 8),), jnp.int32),
            pltpu.HBM((2 * _NS * NP,), jnp.float32),
            pltpu.SemaphoreType.DMA,
            pltpu.SemaphoreType.DMA,
        ],
    )(x_flat, edge_index, attrs_flat, ainv_tab, zfac_tab, rminv_tab)

    pr = partials.reshape(2, NP // 128, 128)
    rows = NP // 128

    def add_body(a_ref, o_ref):
        o_ref[...] = a_ref[0] + a_ref[1]

    blk = 80 if rows % 80 == 0 else 8
    out = pl.pallas_call(
        add_body,
        grid=(rows // blk,),
        in_specs=[pl.BlockSpec((2, blk, 128), lambda i: (0, i, 0))],
        out_specs=pl.BlockSpec((blk, 128), lambda i: (i, 0)),
        out_shape=jax.ShapeDtypeStruct((rows, 128), jnp.float32),
    )(pr)
    return out.reshape(NP)[:n_real]


def kernel(x, node_attrs, edge_index, atomic_numbers):
    N, NE = node_attrs.shape
    E = x.shape[0]

    z = atomic_numbers.astype(jnp.float32)
    pw = jnp.power(z, jnp.float32(0.3))
    radii = jnp.asarray(_COVALENT_RADII, jnp.float32)
    rad = radii[atomic_numbers]
    ii = jnp.minimum(jnp.arange(16), NE - 1)
    pwp = pw[ii]
    zp = z[ii]
    radp = rad[ii]
    ainv_tab = ((pwp[:, None] + pwp[None, :]) / (0.4543 * 0.529)).astype(jnp.float32).reshape(256)[:160]
    zfac_tab = (0.5 * 14.3996 * zp[:, None] * zp[None, :]).astype(jnp.float32).reshape(256)[:160]
    rminv_tab = (1.0 / (radp[:, None] + radp[None, :])).astype(jnp.float32).reshape(256)[:160]

    attrs_in = node_attrs
    if N % _BR:
        attrs_in = jnp.pad(node_attrs, ((0, _BR - N % _BR), (0, 0)))
    NP = _round_up(_round_up(N, _BR), _NW * 640)

    return _zbl_sc(x, edge_index, attrs_in, ainv_tab,
                   zfac_tab, rminv_tab, n_real=N, np_pad=NP)
